# baseline (speedup 1.0000x reference)
import functools

import jax
import jax.numpy as jnp
from jax import lax
from jax.experimental import pallas as pl
from jax.experimental.pallas import tpu as pltpu
from jax.experimental.pallas import tpu_sc as plsc

VOCAB = 1000000
EMB = 64
CTX = 200

NUM_WORKERS = 32
ROWS_PER_WORKER = 8
ACTIVE_WORKERS = CTX // ROWS_PER_WORKER

BK = 16384


def _sc_gather_mean(context, emb_table):
    mesh = plsc.VectorSubcoreMesh(core_axis_name="c", subcore_axis_name="s")

    @functools.partial(
        pl.kernel,
        mesh=mesh,
        out_type=jax.ShapeDtypeStruct((NUM_WORKERS * EMB,), jnp.float32),
        scratch_types=[
            pltpu.VMEM((ROWS_PER_WORKER,), jnp.int32),
            pltpu.VMEM((ROWS_PER_WORKER, EMB), jnp.float32),
            pltpu.VMEM((EMB,), jnp.float32),
            pltpu.SemaphoreType.DMA,
        ],
        compiler_params=pltpu.CompilerParams(use_tc_tiling_on_sc=False),
    )
    def sc_kernel(ctx_hbm, emb_hbm, out_hbm, idx_v, rows_v, acc_v, sem):
        nc = 2
        wid = lax.axis_index("s") * nc + lax.axis_index("c")

        @pl.when(wid < ACTIVE_WORKERS)
        def _():
            pltpu.sync_copy(ctx_hbm.at[pl.ds(wid * ROWS_PER_WORKER,
                                             ROWS_PER_WORKER)], idx_v)
            pltpu.async_copy(emb_hbm.at[idx_v], rows_v, sem).wait()
            for c in range(EMB // 16):
                s = rows_v[0, pl.ds(c * 16, 16)]
                for r in range(1, ROWS_PER_WORKER):
                    s = s + rows_v[r, pl.ds(c * 16, 16)]
                acc_v[pl.ds(c * 16, 16)] = s * (1.0 / CTX)

        @pl.when(wid >= ACTIVE_WORKERS)
        def _():
            for c in range(EMB // 16):
                acc_v[pl.ds(c * 16, 16)] = jnp.zeros((16,), jnp.float32)

        pltpu.sync_copy(acc_v, out_hbm.at[pl.ds(wid * EMB, EMB)])

    return sc_kernel(context, emb_table)


def _tc_body(p_ref, w_ref, b_ref, o_ref):
    avg = jnp.sum(p_ref[...], axis=0, keepdims=True)
    o_ref[...] = lax.dot_general(
        avg, w_ref[...], (((1,), (1,)), ((), ())),
        preferred_element_type=jnp.float32) + b_ref[...]


def kernel(context, emb_table, W, b):
    partial = _sc_gather_mean(context.astype(jnp.int32), emb_table)
    psum = partial.reshape(NUM_WORKERS, EMB)
    out = pl.pallas_call(
        _tc_body,
        grid=(pl.cdiv(VOCAB, BK),),
        in_specs=[
            pl.BlockSpec((NUM_WORKERS, EMB), lambda i: (0, 0)),
            pl.BlockSpec((BK, EMB), lambda i: (i, 0)),
            pl.BlockSpec((1, BK), lambda i: (0, i)),
        ],
        out_specs=pl.BlockSpec((1, BK), lambda i: (0, i)),
        out_shape=jax.ShapeDtypeStruct((1, VOCAB), jnp.float32),
    )(psum, W, b.reshape(1, VOCAB))
    return out


# speedup vs baseline: 8.9677x; 8.9677x over previous
import jax
import jax.numpy as jnp
from jax import lax
from jax.experimental import pallas as pl
from jax.experimental.pallas import tpu as pltpu

VOCAB = 1000000
EMB = 64
CTX = 200

IPB = 4
BK = 32768


def _gather_body(ctx_ref, *refs):
    blk_refs, o_ref = refs[:IPB], refs[IPB]
    i = pl.program_id(0)

    @pl.when(i == 0)
    def _():
        o_ref[...] = jnp.zeros_like(o_ref)

    lanes = lax.broadcasted_iota(jnp.int32, (1, 128), 1)
    acc = o_ref[...]
    for k in range(IPB):
        lane = ctx_ref[IPB * i + k] % 128
        acc += blk_refs[k][...] * (lanes == lane).astype(jnp.float32)
    o_ref[...] = acc


def _matvec_body(p_ref, w_ref, b_ref, o_ref):
    avg_col = jnp.sum(p_ref[...], axis=1, keepdims=True) * (1.0 / CTX)
    o_ref[...] = lax.dot_general(
        avg_col, w_ref[...], (((0,), (0,)), ((), ())),
        preferred_element_type=jnp.float32) + b_ref[...]


def kernel(context, emb_table, W, b):
    emb_t = emb_table.T
    w_t = W.T

    def _blk_spec(k):
        return pl.BlockSpec(
            (EMB, 128), lambda i, c, k=k: (0, c[IPB * i + k] // 128))

    acc128 = pl.pallas_call(
        _gather_body,
        grid_spec=pltpu.PrefetchScalarGridSpec(
            num_scalar_prefetch=1,
            grid=(CTX // IPB,),
            in_specs=[_blk_spec(k) for k in range(IPB)],
            out_specs=pl.BlockSpec((EMB, 128), lambda i, c: (0, 0)),
        ),
        out_shape=jax.ShapeDtypeStruct((EMB, 128), jnp.float32),
    )(context.astype(jnp.int32), *([emb_t] * IPB))

    out = pl.pallas_call(
        _matvec_body,
        grid=(pl.cdiv(VOCAB, BK),),
        in_specs=[
            pl.BlockSpec((EMB, 128), lambda i: (0, 0)),
            pl.BlockSpec((EMB, BK), lambda i: (0, i)),
            pl.BlockSpec((1, BK), lambda i: (0, i)),
        ],
        out_specs=pl.BlockSpec((1, BK), lambda i: (0, i)),
        out_shape=jax.ShapeDtypeStruct((1, VOCAB), jnp.float32),
    )(acc128, w_t, b.reshape(1, VOCAB))
    return out


# speedup vs baseline: 11.5865x; 1.2920x over previous
import jax
import jax.numpy as jnp
from jax import lax
from jax.experimental import pallas as pl
from jax.experimental.pallas import tpu as pltpu

VOCAB = 1000000
EMB = 64
CTX = 200

IPB = 8
BK = 65536


def _gather_body(ctx_ref, *refs):
    blk_refs, o_ref = refs[:IPB], refs[IPB]
    i = pl.program_id(0)

    @pl.when(i == 0)
    def _():
        o_ref[...] = jnp.zeros_like(o_ref)

    lanes = lax.broadcasted_iota(jnp.int32, (1, 128), 1)
    acc = o_ref[...]
    for k in range(IPB):
        lane = ctx_ref[IPB * i + k] % 128
        acc += blk_refs[k][...] * (lanes == lane).astype(jnp.float32)
    o_ref[...] = acc


def _matvec_body(p_ref, w_ref, b_ref, o_ref):
    avg_col = jnp.sum(p_ref[...], axis=1, keepdims=True) * (1.0 / CTX)
    o_ref[...] = lax.dot_general(
        avg_col, w_ref[...], (((0,), (0,)), ((), ())),
        preferred_element_type=jnp.float32) + b_ref[...].reshape(1, -1)


def kernel(context, emb_table, W, b):
    emb_t = emb_table.T
    w_t = W.T

    def _blk_spec(k):
        return pl.BlockSpec(
            (EMB, 128), lambda i, c, k=k: (0, c[IPB * i + k] // 128))

    acc128 = pl.pallas_call(
        _gather_body,
        grid_spec=pltpu.PrefetchScalarGridSpec(
            num_scalar_prefetch=1,
            grid=(CTX // IPB,),
            in_specs=[_blk_spec(k) for k in range(IPB)],
            out_specs=pl.BlockSpec((EMB, 128), lambda i, c: (0, 0)),
        ),
        out_shape=jax.ShapeDtypeStruct((EMB, 128), jnp.float32),
    )(context.astype(jnp.int32), *([emb_t] * IPB))

    out = pl.pallas_call(
        _matvec_body,
        grid=(pl.cdiv(VOCAB, BK),),
        in_specs=[
            pl.BlockSpec((EMB, 128), lambda i: (0, 0)),
            pl.BlockSpec((EMB, BK), lambda i: (0, i)),
            pl.BlockSpec((BK,), lambda i: (i,)),
        ],
        out_specs=pl.BlockSpec((1, BK), lambda i: (0, i)),
        out_shape=jax.ShapeDtypeStruct((1, VOCAB), jnp.float32),
    )(acc128, w_t, b)
    return out


# speedup vs baseline: 12.6889x; 1.0951x over previous
import jax
import jax.numpy as jnp
from jax import lax
from jax.experimental import pallas as pl
from jax.experimental.pallas import tpu as pltpu

VOCAB = 1000000
EMB = 64
CTX = 200

IPB = 20
BK = 32768


def _gather_body(ctx_ref, *refs):
    blk_refs, o_ref = refs[:IPB], refs[IPB]
    i = pl.program_id(0)

    @pl.when(i == 0)
    def _():
        o_ref[...] = jnp.zeros_like(o_ref)

    lanes = lax.broadcasted_iota(jnp.int32, (1, 128), 1)
    acc = o_ref[...]
    for k in range(IPB):
        lane = ctx_ref[IPB * i + k] % 128
        acc += blk_refs[k][...] * (lanes == lane).astype(jnp.float32)
    o_ref[...] = acc


def _matvec_body(p_ref, w_ref, b_ref, o_ref):
    avg_col = jnp.sum(p_ref[...], axis=1, keepdims=True) * (1.0 / CTX)
    o_ref[...] = lax.dot_general(
        avg_col, w_ref[...], (((0,), (0,)), ((), ())),
        preferred_element_type=jnp.float32) + b_ref[...].reshape(1, -1)


def kernel(context, emb_table, W, b):
    emb_t = emb_table.T
    w_t = W.T

    def _blk_spec(k):
        return pl.BlockSpec(
            (EMB, 128), lambda i, c, k=k: (0, c[IPB * i + k] // 128))

    acc128 = pl.pallas_call(
        _gather_body,
        grid_spec=pltpu.PrefetchScalarGridSpec(
            num_scalar_prefetch=1,
            grid=(CTX // IPB,),
            in_specs=[_blk_spec(k) for k in range(IPB)],
            out_specs=pl.BlockSpec((EMB, 128), lambda i, c: (0, 0)),
        ),
        out_shape=jax.ShapeDtypeStruct((EMB, 128), jnp.float32),
    )(context.astype(jnp.int32), *([emb_t] * IPB))

    out = pl.pallas_call(
        _matvec_body,
        grid=(pl.cdiv(VOCAB, BK),),
        in_specs=[
            pl.BlockSpec((EMB, 128), lambda i: (0, 0)),
            pl.BlockSpec((EMB, BK), lambda i: (0, i)),
            pl.BlockSpec((BK,), lambda i: (i,)),
        ],
        out_specs=pl.BlockSpec((1, BK), lambda i: (0, i)),
        out_shape=jax.ShapeDtypeStruct((1, VOCAB), jnp.float32),
    )(acc128, w_t, b)
    return out
